# baseline (speedup 1.0000x reference)
import functools

import jax
import jax.numpy as jnp
from jax import lax
from jax.experimental import pallas as pl
from jax.experimental.pallas import tpu as pltpu
from jax.experimental.pallas import tpu_sc as plsc

_H4 = 262144
_BLKN = 8192


def _pack4(tableT):
    D = tableT.shape[0]
    nb = _H4 // _BLKN
    last = (tableT.shape[1] - 1) // _BLKN
    dn = (((0,), (0,)), ((), ()))

    def rnd16(u):
        bits = lax.bitcast_convert_type(u, jnp.uint32)
        return (bits + jnp.uint32(0x8000)) & jnp.uint32(0xFFFF0000)

    def body(x0_ref, x1_ref, x2_ref, x3_ref, eye_ref, o_ref):
        eye = eye_ref[...]
        u = [
            lax.dot_general(x_ref[...], eye, dn, preferred_element_type=jnp.float32)
            for x_ref in (x0_ref, x1_ref, x2_ref, x3_ref)
        ]
        w01 = rnd16(u[0]) | jnp.right_shift(rnd16(u[1]), 16)
        w23 = rnd16(u[2]) | jnp.right_shift(rnd16(u[3]), 16)
        w = jnp.concatenate([w01, w23], axis=1)
        o_ref[...] = lax.bitcast_convert_type(w, jnp.int32)

    return pl.pallas_call(
        body,
        grid=(nb,),
        in_specs=[
            pl.BlockSpec((D, _BLKN), lambda i: (0, i)),
            pl.BlockSpec((D, _BLKN), lambda i: (0, jnp.minimum(i + nb, last))),
            pl.BlockSpec((D, _BLKN), lambda i: (0, jnp.minimum(i + 2 * nb, last))),
            pl.BlockSpec((D, _BLKN), lambda i: (0, jnp.minimum(i + 3 * nb, last))),
            pl.BlockSpec((D, D), lambda i: (0, 0)),
        ],
        out_specs=pl.BlockSpec((_BLKN, 2 * D), lambda i: (i, 0)),
        out_shape=jax.ShapeDtypeStruct((_H4, 2 * D), jnp.int32),
    )(tableT, tableT, tableT, tableT, jnp.eye(D, dtype=jnp.float32))


def _sc_gather(table4, idx4):
    B = idx4.shape[0]
    D2 = table4.shape[1]
    info = plsc.get_sparse_core_info()
    NC, NS = info.num_cores, info.num_subcores
    NW = NC * NS
    b_per_w = B // NW

    mesh = plsc.VectorSubcoreMesh(core_axis_name="c", subcore_axis_name="s")

    @functools.partial(
        pl.kernel,
        mesh=mesh,
        out_type=jax.ShapeDtypeStruct((B, D2), jnp.int32),
        scratch_types=[
            pltpu.VMEM((b_per_w,), jnp.int32),
            pltpu.VMEM((b_per_w, D2), jnp.int32),
            pltpu.SemaphoreType.DMA,
        ],
    )
    def gather_kernel(table_hbm, idx_hbm, out_hbm, idx_v, rows_v, sem):
        wid = lax.axis_index("s") * NC + lax.axis_index("c")
        base = wid * b_per_w
        pltpu.sync_copy(idx_hbm.at[pl.ds(base, b_per_w)], idx_v)
        pltpu.async_copy(table_hbm.at[idx_v], rows_v, sem).wait()
        pltpu.sync_copy(rows_v, out_hbm.at[pl.ds(base, b_per_w)])

    return gather_kernel(table4, idx4)


def _mlp_unpack(x4, hi, lo, W1, b1, W2, b2):
    B = x4.shape[0]
    D = W1.shape[0]
    H1 = W1.shape[1]
    H2 = W2.shape[1]
    BLK = 2048

    def body(x_ref, hi_ref, lo_ref, w1_ref, b1_ref, w2_ref, b2_ref, o_ref):
        x = x_ref[...]
        xa = jnp.where(hi_ref[...] > 0, x[:, D:], x[:, :D])
        bits = jnp.where(lo_ref[...] > 0, jnp.left_shift(xa, 16), xa) & (-65536)
        e = lax.bitcast_convert_type(bits, jnp.float32)
        h = jnp.dot(e, w1_ref[...], preferred_element_type=jnp.float32)
        h = jnp.maximum(h + b1_ref[...], 0.0)
        o = jnp.dot(h, w2_ref[...], preferred_element_type=jnp.float32)
        o_ref[...] = o + b2_ref[...]

    return pl.pallas_call(
        body,
        grid=(B // BLK,),
        in_specs=[
            pl.BlockSpec((BLK, 2 * D), lambda i: (i, 0)),
            pl.BlockSpec((BLK, 1), lambda i: (i, 0)),
            pl.BlockSpec((BLK, 1), lambda i: (i, 0)),
            pl.BlockSpec((D, H1), lambda i: (0, 0)),
            pl.BlockSpec((1, H1), lambda i: (0, 0)),
            pl.BlockSpec((H1, H2), lambda i: (0, 0)),
            pl.BlockSpec((1, H2), lambda i: (0, 0)),
        ],
        out_specs=pl.BlockSpec((BLK, H2), lambda i: (i, 0)),
        out_shape=jax.ShapeDtypeStruct((B, H2), jnp.float32),
    )(x4, hi, lo, W1, b1.reshape(1, H1), W2, b2.reshape(1, H2))


def kernel(inputs, table, W1, b1, W2, b2):
    idx = inputs.astype(jnp.int32)
    table4 = _pack4(table.T)
    half = idx // _H4
    idx4 = idx - half * _H4
    hi = (half >= 2).astype(jnp.int32).reshape(-1, 1)
    lo = (half % 2).astype(jnp.int32).reshape(-1, 1)
    x4 = _sc_gather(table4, idx4)
    return _mlp_unpack(x4, hi, lo, W1, b1, W2, b2)


# speedup vs baseline: 1.1476x; 1.1476x over previous
import functools

import jax
import jax.numpy as jnp
from jax import lax
from jax.experimental import pallas as pl
from jax.experimental.pallas import tpu as pltpu
from jax.experimental.pallas import tpu_sc as plsc

_H = 524288
_BLKN = 16384


def _pack_halves(tableT):
    D = tableT.shape[0]
    nb = _H // _BLKN
    last = (tableT.shape[1] - 1) // _BLKN

    dn = (((0,), (0,)), ((), ()))

    def body(x0_ref, x1_ref, e0_ref, e1_ref, o_ref):
        x0 = x0_ref[...].astype(jnp.bfloat16)
        x1 = x1_ref[...].astype(jnp.bfloat16)
        t0 = lax.dot_general(x0, e0_ref[...], dn, preferred_element_type=jnp.float32)
        t1 = lax.dot_general(x1, e1_ref[...], dn, preferred_element_type=jnp.float32)
        o_ref[...] = t0 + t1

    eye = jnp.eye(D, dtype=jnp.bfloat16)
    zero = jnp.zeros((D, D), dtype=jnp.bfloat16)
    e0 = jnp.concatenate([eye, zero], axis=1)
    e1 = jnp.concatenate([zero, eye], axis=1)
    return pl.pallas_call(
        body,
        grid=(nb,),
        in_specs=[
            pl.BlockSpec((D, _BLKN), lambda i: (0, i)),
            pl.BlockSpec((D, _BLKN), lambda i: (0, jnp.minimum(i + nb, last))),
            pl.BlockSpec((D, 2 * D), lambda i: (0, 0)),
            pl.BlockSpec((D, 2 * D), lambda i: (0, 0)),
        ],
        out_specs=pl.BlockSpec((_BLKN, 2 * D), lambda i: (i, 0)),
        out_shape=jax.ShapeDtypeStruct((_H, 2 * D), jnp.float32),
    )(tableT, tableT, e0, e1)


def _sc_gather(table2, idx2):
    B = idx2.shape[0]
    D2 = table2.shape[1]
    info = plsc.get_sparse_core_info()
    NC, NS = info.num_cores, info.num_subcores
    NW = NC * NS
    b_per_w = B // NW

    mesh = plsc.VectorSubcoreMesh(core_axis_name="c", subcore_axis_name="s")

    @functools.partial(
        pl.kernel,
        mesh=mesh,
        out_type=jax.ShapeDtypeStruct((B, D2), jnp.float32),
        scratch_types=[
            pltpu.VMEM((b_per_w,), jnp.int32),
            pltpu.VMEM((b_per_w, D2), jnp.float32),
            pltpu.SemaphoreType.DMA,
        ],
    )
    def gather_kernel(table_hbm, idx_hbm, out_hbm, idx_v, rows_v, sem):
        wid = lax.axis_index("s") * NC + lax.axis_index("c")
        base = wid * b_per_w
        pltpu.sync_copy(idx_hbm.at[pl.ds(base, b_per_w)], idx_v)
        pltpu.async_copy(table_hbm.at[idx_v], rows_v, sem).wait()
        pltpu.sync_copy(rows_v, out_hbm.at[pl.ds(base, b_per_w)])

    return gather_kernel(table2, idx2)


def _mlp_select(x2, par, W1, b1, W2, b2):
    B = x2.shape[0]
    D = W1.shape[0]
    H1 = W1.shape[1]
    H2 = W2.shape[1]
    BLK = 2048

    dn_t = (((1,), (1,)), ((), ()))

    def body(x_ref, p_ref, w1_ref, b1_ref, w2_ref, b2_ref, eye_ref, o_ref):
        x = x_ref[...]
        e = jnp.where(p_ref[...] > 0, x[:, D:], x[:, :D])
        h = jnp.dot(e, w1_ref[...], preferred_element_type=jnp.float32)
        h = jnp.maximum(h + b1_ref[...], 0.0)
        o = jnp.dot(h, w2_ref[...], preferred_element_type=jnp.float32)
        o = o + b2_ref[...]
        o_ref[...] = lax.dot_general(
            eye_ref[...], o, dn_t, preferred_element_type=jnp.float32
        )

    return pl.pallas_call(
        body,
        grid=(B // BLK,),
        in_specs=[
            pl.BlockSpec((BLK, 2 * D), lambda i: (i, 0)),
            pl.BlockSpec((BLK, 1), lambda i: (i, 0)),
            pl.BlockSpec((D, H1), lambda i: (0, 0)),
            pl.BlockSpec((1, H1), lambda i: (0, 0)),
            pl.BlockSpec((H1, H2), lambda i: (0, 0)),
            pl.BlockSpec((1, H2), lambda i: (0, 0)),
            pl.BlockSpec((H2, H2), lambda i: (0, 0)),
        ],
        out_specs=pl.BlockSpec((H2, BLK), lambda i: (0, i)),
        out_shape=jax.ShapeDtypeStruct((H2, B), jnp.float32),
    )(x2, par, W1, b1.reshape(1, H1), W2, b2.reshape(1, H2),
      jnp.eye(H2, dtype=jnp.float32))


def kernel(inputs, table, W1, b1, W2, b2):
    idx = inputs.astype(jnp.int32)
    table2 = _pack_halves(table.T)
    par = (idx >= _H).astype(jnp.int32)
    idx2 = idx - _H * par
    e2 = _sc_gather(table2, idx2)
    return _mlp_select(e2, par.reshape(-1, 1), W1, b1, W2, b2).T
